# Initial kernel scaffold; baseline (speedup 1.0000x reference)
import jax
import jax.numpy as jnp
from jax.experimental import pallas as pl


def kernel(x_cat, tables):
    raise NotImplementedError("write your pallas kernel here")


# speedup vs baseline: 1.1448x; 1.1448x over previous
import functools

import jax
import jax.numpy as jnp
from jax import lax
from jax.experimental import pallas as pl
from jax.experimental.pallas import tpu as pltpu
from jax.experimental.pallas import tpu_sc as plsc

NUM_FIELDS = 26
VOCAB = 100000
EMBED_DIM = 32
BATCH = 16384

NC = 2
NS = 16
NW = NC * NS
L = 16

B_TOT = BATCH * NUM_FIELDS
GCH = 128
CPW = B_TOT // (NW * GCH)
BPW = CPW * GCH
KPG = 8
RCH = KPG * GCH
NR = CPW // KPG
OFFP = 208


def _body(xcat_hbm, tab_hbm, out_hbm, idx_v, off_v, buf_v, gsem, wsem):
    wid = lax.axis_index("s") * NC + lax.axis_index("c")
    row0 = wid * CPW

    pltpu.sync_copy(xcat_hbm.at[pl.ds(row0, CPW)], idx_v)

    for k in range(OFFP // L):
        j = lax.iota(jnp.int32, L) + (k * L)
        off_v[pl.ds(k * L, L)] = lax.rem(j, NUM_FIELDS) * VOCAB

    def fix_row(g, _):
        for q in range(GCH // L):
            ob = lax.rem(g * GCH + q * L, OFFP)
            idx_v[g, pl.ds(q * L, L)] = (
                idx_v[g, pl.ds(q * L, L)] + off_v[pl.ds(ob, L)]
            )
        return 0

    lax.fori_loop(0, CPW, fix_row, 0)

    for r in range(NR):
        cps = []
        for j in range(KPG):
            g = r * KPG + j
            cps.append(
                pltpu.async_copy(
                    tab_hbm.at[idx_v.at[g]],
                    buf_v.at[pl.ds(j * GCH, GCH)],
                    gsem,
                )
            )
        for cp in cps:
            cp.wait()
        pltpu.sync_copy(
            buf_v, out_hbm.at[pl.ds(wid * BPW + r * RCH, RCH)]
        )


@jax.jit
def _embed(xcat2d, tab_flat):
    mesh = plsc.VectorSubcoreMesh(core_axis_name="c", subcore_axis_name="s")
    f = pl.kernel(
        _body,
        out_type=jax.ShapeDtypeStruct((B_TOT, EMBED_DIM), jnp.float32),
        mesh=mesh,
        scratch_types=[
            pltpu.VMEM((CPW, GCH), jnp.int32),
            pltpu.VMEM((OFFP,), jnp.int32),
            pltpu.VMEM((RCH, EMBED_DIM), jnp.float32),
            pltpu.SemaphoreType.DMA,
            pltpu.SemaphoreType.DMA,
        ],
        compiler_params=pltpu.CompilerParams(use_tc_tiling_on_sc=False),
    )
    return f(xcat2d, tab_flat)


def kernel(x_cat, tables):
    xcat2d = x_cat.astype(jnp.int32).reshape(B_TOT // GCH, GCH)
    tab_flat = tables.reshape(NUM_FIELDS * VOCAB, EMBED_DIM)
    out = _embed(xcat2d, tab_flat)
    return out.reshape(BATCH, NUM_FIELDS, EMBED_DIM)
